# Initial kernel scaffold; baseline (speedup 1.0000x reference)
import jax
import jax.numpy as jnp
from jax.experimental import pallas as pl


def kernel(x, edge_index, edge_attr, gcn_W, gcn_b, W_ih, b_ih, W_hh, b_hh, lin1_W, lin1_b, lin2_W, lin2_b):
    raise NotImplementedError("write your pallas kernel here")


# speedup vs baseline: 23.8623x; 23.8623x over previous
import functools

import jax
import jax.numpy as jnp
from jax import lax
from jax.experimental import pallas as pl
from jax.experimental.pallas import tpu as pltpu
from jax.experimental.pallas import tpu_sc as plsc

N = 1024
D = 1024
E = 65536

NC = 2
NS = 16
EC = E // NC
ET = EC // NS
GROUP = 128
NGROUP = ET // GROUP
ZCH = 2048
SLICE = (N * N) // NS


def _adj_body(src_hbm, dst_hbm, out_hbm, srcv, dstv, fidx, onesv, zbuf, a_sh):
    c = lax.axis_index("c")
    s = lax.axis_index("s")

    for k in range(GROUP // 16):
        onesv[pl.ds(k * 16, 16)] = jnp.ones((16,), jnp.float32)
    for k in range(ZCH // 16):
        zbuf[pl.ds(k * 16, 16)] = jnp.zeros((16,), jnp.float32)

    def _zero(i, carry):
        pltpu.sync_copy(zbuf, a_sh.at[pl.ds(s * SLICE + i * ZCH, ZCH)])
        return carry
    lax.fori_loop(0, SLICE // ZCH, _zero, 0)

    base = (c * NS + s) * ET
    pltpu.sync_copy(src_hbm.at[pl.ds(base, ET)], srcv)
    pltpu.sync_copy(dst_hbm.at[pl.ds(base, ET)], dstv)

    for g in range(NGROUP):
        for k in range(GROUP // 16):
            off = g * GROUP + k * 16
            sv = srcv[pl.ds(off, 16)]
            dv = dstv[pl.ds(off, 16)]
            fidx[g, pl.ds(k * 16, 16)] = dv * N + sv

    plsc.subcore_barrier()

    for g in range(NGROUP):
        pltpu.sync_copy(onesv, a_sh.at[fidx.at[g]], add=True)

    plsc.subcore_barrier()

    out_off = (c * NS + s) * SLICE
    pltpu.sync_copy(a_sh.at[pl.ds(s * SLICE, SLICE)], out_hbm.at[pl.ds(out_off, SLICE)])


_adj_kernel = functools.partial(
    pl.kernel,
    out_type=jax.ShapeDtypeStruct((NC * N * N,), jnp.float32),
    mesh=plsc.VectorSubcoreMesh(
        core_axis_name="c", subcore_axis_name="s", num_cores=NC, num_subcores=NS
    ),
    scratch_types=[
        pltpu.VMEM((ET,), jnp.int32),
        pltpu.VMEM((ET,), jnp.int32),
        pltpu.VMEM((NGROUP, GROUP), jnp.int32),
        pltpu.VMEM((GROUP,), jnp.float32),
        pltpu.VMEM((ZCH,), jnp.float32),
        pltpu.VMEM_SHARED((N * N,), jnp.float32),
    ],
)(_adj_body)


def _sigmoid(t):
    return 1.0 / (1.0 + jnp.exp(-t))


def _tc_body(a_ref, x_ref, gw_ref, gb_ref, wih_ref, bih_ref, whh_ref, bhh_ref,
             l1w_ref, l1b_ref, l2w_ref, l2b_ref, q_ref):
    acnt = a_ref[0] + a_ref[1]
    deg = jnp.sum(acnt, axis=1) + 1.0
    dinv = lax.rsqrt(deg)

    x = x_ref[...]
    xw = jnp.dot(x, gw_ref[...], preferred_element_type=jnp.float32)
    xs = xw * dinv[:, None]
    agg = (jnp.dot(acnt, xs, preferred_element_type=jnp.float32) + xs) * dinv[:, None]
    nf = jnp.maximum(agg + gb_ref[...][None, :], 0.0)

    bih = bih_ref[...]
    bhh = bhh_ref[...]

    def gate(inp, w_ref, lo):
        w = w_ref[pl.ds(lo, D), :]
        return lax.dot_general(inp, w, (((1,), (1,)), ((), ())),
                               preferred_element_type=jnp.float32)

    r = _sigmoid(gate(nf, wih_ref, 0) + bih[:D][None, :]
                 + gate(x, whh_ref, 0) + bhh[:D][None, :])
    z = _sigmoid(gate(nf, wih_ref, D) + bih[D:2 * D][None, :]
                 + gate(x, whh_ref, D) + bhh[D:2 * D][None, :])
    i_n = gate(nf, wih_ref, 2 * D) + bih[2 * D:][None, :]
    h_n = gate(x, whh_ref, 2 * D) + bhh[2 * D:][None, :]
    nt = jnp.tanh(i_n + r * h_n)
    h = (1.0 - z) * nt + z * x

    srow = jnp.sum(h, axis=1)[None, :]
    h1 = jnp.maximum(
        lax.dot_general(srow, l1w_ref[...], (((1,), (1,)), ((), ())),
                        preferred_element_type=jnp.float32)
        + l1b_ref[...][None, :], 0.0)
    q_ref[...] = jnp.full((1, 1), jnp.sum(h1 * l2w_ref[...]) + l2b_ref[0],
                          jnp.float32)


def kernel(x, edge_index, edge_attr, gcn_W, gcn_b, W_ih, b_ih, W_hh, b_hh,
           lin1_W, lin1_b, lin2_W, lin2_b):
    del edge_attr
    src = edge_index[0]
    dst = edge_index[1]
    a_flat = _adj_kernel(src, dst)
    a2 = a_flat.reshape(NC, N, N)
    q = pl.pallas_call(
        _tc_body,
        out_shape=jax.ShapeDtypeStruct((1, 1), jnp.float32),
    )(a2, x, gcn_W, gcn_b, W_ih, b_ih, W_hh, b_hh, lin1_W, lin1_b, lin2_W, lin2_b)
    return q.reshape(1)


# speedup vs baseline: 28.4902x; 1.1939x over previous
import jax
import jax.numpy as jnp
from jax import lax
from jax.experimental import pallas as pl
from jax.experimental.pallas import tpu as pltpu
from jax.experimental.pallas import tpu_sc as plsc

N = 1024
D = 1024
E = 65536

NC = 2
NS = 16
HALF = N // NC
HW = HALF * N
SL = HW // NS
EPS = E // NS
GROUP = 128
NG = EPS // GROUP
ZCH = 2048
FIRE = 16


def _adj_body(ei_hbm, out_hbm, srcv, dstv, fidx, vals, zbuf, a_sh, zsem, esem, ssem):
    c = lax.axis_index("c")
    s = lax.axis_index("s")

    base = s * EPS
    eld = [
        pltpu.async_copy(ei_hbm.at[0, pl.ds(base, EPS)], srcv, esem),
        pltpu.async_copy(ei_hbm.at[1, pl.ds(base, EPS)], dstv, esem),
    ]

    def _fill_zero(i, carry):
        zbuf[pl.ds(i * 16, 16)] = jnp.zeros((16,), jnp.float32)
        return carry
    lax.fori_loop(0, ZCH // 16, _fill_zero, 0)

    zld = [
        pltpu.async_copy(zbuf, a_sh.at[pl.ds(s * SL + i * ZCH, ZCH)], zsem)
        for i in range(SL // ZCH)
    ]
    for dsc in eld:
        dsc.wait()

    def _grp(g, carry):
        for k in range(GROUP // 16):
            off = g * GROUP + k * 16
            sv = srcv[pl.ds(off, 16)]
            dv = dstv[pl.ds(off, 16)]
            mine = lax.shift_right_logical(dv, 9) == c
            loc = ((dv - c * HALF) * N + sv) & (HW - 1)
            fidx[g, pl.ds(k * 16, 16)] = loc
            vals[g, pl.ds(k * 16, 16)] = jnp.where(mine, 1.0, 0.0)
        return carry
    lax.fori_loop(0, NG, _grp, 0)

    for dsc in zld:
        dsc.wait()
    plsc.subcore_barrier()

    for lo in range(0, NG, FIRE):
        descs = [
            pltpu.async_copy(vals.at[g], a_sh.at[fidx.at[g]], ssem, add=True)
            for g in range(lo, lo + FIRE)
        ]
        for dsc in descs:
            dsc.wait()

    plsc.subcore_barrier()

    out_off = c * HW + s * SL
    pltpu.sync_copy(a_sh.at[pl.ds(s * SL, SL)], out_hbm.at[pl.ds(out_off, SL)])


_adj_kernel = pl.kernel(
    _adj_body,
    out_type=jax.ShapeDtypeStruct((N * N,), jnp.float32),
    mesh=plsc.VectorSubcoreMesh(
        core_axis_name="c", subcore_axis_name="s", num_cores=NC, num_subcores=NS
    ),
    scratch_types=[
        pltpu.VMEM((EPS,), jnp.int32),
        pltpu.VMEM((EPS,), jnp.int32),
        pltpu.VMEM((NG, GROUP), jnp.int32),
        pltpu.VMEM((NG, GROUP), jnp.float32),
        pltpu.VMEM((ZCH,), jnp.float32),
        pltpu.VMEM_SHARED((HW,), jnp.float32),
        pltpu.SemaphoreType.DMA,
        pltpu.SemaphoreType.DMA,
        pltpu.SemaphoreType.DMA,
    ],
)


def _sigmoid(t):
    return 1.0 / (1.0 + jnp.exp(-t))


def _tc_body(a_ref, x_ref, gw_ref, gb_ref, wih_ref, bih_ref, whh_ref, bhh_ref,
             l1w_ref, l1b_ref, l2w_ref, l2b_ref, q_ref):
    acnt = a_ref[...]
    deg = jnp.sum(acnt, axis=1) + 1.0
    dinv = lax.rsqrt(deg)

    x = x_ref[...]
    xw = jnp.dot(x, gw_ref[...], preferred_element_type=jnp.float32)
    xs = xw * dinv[:, None]
    agg = (jnp.dot(acnt, xs, preferred_element_type=jnp.float32) + xs) * dinv[:, None]
    nf = jnp.maximum(agg + gb_ref[...][None, :], 0.0)

    bih = bih_ref[...]
    bhh = bhh_ref[...]

    def gate(inp, w_ref, lo):
        w = w_ref[pl.ds(lo, D), :]
        return lax.dot_general(inp, w, (((1,), (1,)), ((), ())),
                               preferred_element_type=jnp.float32)

    r = _sigmoid(gate(nf, wih_ref, 0) + bih[:D][None, :]
                 + gate(x, whh_ref, 0) + bhh[:D][None, :])
    z = _sigmoid(gate(nf, wih_ref, D) + bih[D:2 * D][None, :]
                 + gate(x, whh_ref, D) + bhh[D:2 * D][None, :])
    i_n = gate(nf, wih_ref, 2 * D) + bih[2 * D:][None, :]
    h_n = gate(x, whh_ref, 2 * D) + bhh[2 * D:][None, :]
    nt = jnp.tanh(i_n + r * h_n)
    h = (1.0 - z) * nt + z * x

    srow = jnp.sum(h, axis=1)[None, :]
    h1 = jnp.maximum(
        lax.dot_general(srow, l1w_ref[...], (((1,), (1,)), ((), ())),
                        preferred_element_type=jnp.float32)
        + l1b_ref[...][None, :], 0.0)
    q_ref[...] = jnp.full((1, 1), jnp.sum(h1 * l2w_ref[...]) + l2b_ref[0],
                          jnp.float32)


def kernel(x, edge_index, edge_attr, gcn_W, gcn_b, W_ih, b_ih, W_hh, b_hh,
           lin1_W, lin1_b, lin2_W, lin2_b):
    del edge_attr
    a_flat = _adj_kernel(edge_index)
    a2 = a_flat.reshape(N, N)
    q = pl.pallas_call(
        _tc_body,
        out_shape=jax.ShapeDtypeStruct((1, 1), jnp.float32),
    )(a2, x, gcn_W, gcn_b, W_ih, b_ih, W_hh, b_hh, lin1_W, lin1_b, lin2_W, lin2_b)
    return q.reshape(1)


# speedup vs baseline: 31.0782x; 1.0908x over previous
import jax
import jax.numpy as jnp
from jax import lax
from jax.experimental import pallas as pl
from jax.experimental.pallas import tpu as pltpu
from jax.experimental.pallas import tpu_sc as plsc

N = 1024
D = 1024
E = 65536

NC = 2
NS = 16
HALF = N // NC
HW = HALF * N
SL = HW // NS
EPS = E // NS
GROUP = 128
NG = EPS // GROUP
ZCH = 2048
FIRE = 16


def _adj_body(ei_hbm, out_hbm, srcv, dstv, fidx, vals, zbuf, a_sh, zsem, esem, ssem):
    c = lax.axis_index("c")
    s = lax.axis_index("s")

    base = s * EPS
    eld = [
        pltpu.async_copy(ei_hbm.at[0, pl.ds(base, EPS)], srcv, esem),
        pltpu.async_copy(ei_hbm.at[1, pl.ds(base, EPS)], dstv, esem),
    ]

    def _fill_zero(i, carry):
        zbuf[pl.ds(i * 16, 16)] = jnp.zeros((16,), jnp.float32)
        return carry
    lax.fori_loop(0, ZCH // 16, _fill_zero, 0)

    zld = [
        pltpu.async_copy(zbuf, a_sh.at[pl.ds(s * SL + i * ZCH, ZCH)], zsem)
        for i in range(SL // ZCH)
    ]
    for dsc in eld:
        dsc.wait()

    def _grp(g, carry):
        for k in range(GROUP // 16):
            off = g * GROUP + k * 16
            sv = srcv[pl.ds(off, 16)]
            dv = dstv[pl.ds(off, 16)]
            mine = lax.shift_right_logical(dv, 9) == c
            loc = ((dv - c * HALF) * N + sv) & (HW - 1)
            fidx[g, pl.ds(k * 16, 16)] = loc
            vals[g, pl.ds(k * 16, 16)] = jnp.where(mine, 1.0, 0.0)
        return carry
    lax.fori_loop(0, NG, _grp, 0)

    for dsc in zld:
        dsc.wait()
    plsc.subcore_barrier()

    for lo in range(0, NG, FIRE):
        descs = [
            pltpu.async_copy(vals.at[g], a_sh.at[fidx.at[g]], ssem, add=True)
            for g in range(lo, lo + FIRE)
        ]
        for dsc in descs:
            dsc.wait()

    plsc.subcore_barrier()

    row0 = c * HALF + s * (SL // N)
    rdescs = [
        pltpu.async_copy(a_sh.at[pl.ds(s * SL + r * N, N)], out_hbm.at[row0 + r], zsem)
        for r in range(SL // N)
    ]
    for dsc in rdescs:
        dsc.wait()


_adj_kernel = pl.kernel(
    _adj_body,
    out_type=jax.ShapeDtypeStruct((N, N), jnp.float32),
    mesh=plsc.VectorSubcoreMesh(
        core_axis_name="c", subcore_axis_name="s", num_cores=NC, num_subcores=NS
    ),
    scratch_types=[
        pltpu.VMEM((EPS,), jnp.int32),
        pltpu.VMEM((EPS,), jnp.int32),
        pltpu.VMEM((NG, GROUP), jnp.int32),
        pltpu.VMEM((NG, GROUP), jnp.float32),
        pltpu.VMEM((ZCH,), jnp.float32),
        pltpu.VMEM_SHARED((HW,), jnp.float32),
        pltpu.SemaphoreType.DMA,
        pltpu.SemaphoreType.DMA,
        pltpu.SemaphoreType.DMA,
    ],
)


def _sigmoid(t):
    return 1.0 / (1.0 + jnp.exp(-t))


def _tc_body(a_ref, x_ref, gw_ref, gb_ref, wih_ref, bih_ref, whh_ref, bhh_ref,
             l1w_ref, l1b_ref, l2w_ref, l2b_ref, q_ref):
    acnt = a_ref[...]
    deg = jnp.sum(acnt, axis=1) + 1.0
    dinv = lax.rsqrt(deg)

    x = x_ref[...]
    bx = x.astype(jnp.bfloat16)
    xw = jnp.dot(bx, gw_ref[...].astype(jnp.bfloat16),
                 preferred_element_type=jnp.float32)
    xs = xw * dinv[:, None]
    agg = (jnp.dot(acnt.astype(jnp.bfloat16), xs.astype(jnp.bfloat16),
                   preferred_element_type=jnp.float32) + xs) * dinv[:, None]
    nf = jnp.maximum(agg + gb_ref[...][None, :], 0.0)
    bnf = nf.astype(jnp.bfloat16)

    bih = bih_ref[...]
    bhh = bhh_ref[...]

    def gate(inp, w_ref, lo):
        w = w_ref[pl.ds(lo, D), :].astype(jnp.bfloat16)
        return lax.dot_general(inp, w, (((1,), (1,)), ((), ())),
                               preferred_element_type=jnp.float32)

    r = _sigmoid(gate(bnf, wih_ref, 0) + bih[:D][None, :]
                 + gate(bx, whh_ref, 0) + bhh[:D][None, :])
    z = _sigmoid(gate(bnf, wih_ref, D) + bih[D:2 * D][None, :]
                 + gate(bx, whh_ref, D) + bhh[D:2 * D][None, :])
    i_n = gate(bnf, wih_ref, 2 * D) + bih[2 * D:][None, :]
    h_n = gate(bx, whh_ref, 2 * D) + bhh[2 * D:][None, :]
    nt = jnp.tanh(i_n + r * h_n)
    h = (1.0 - z) * nt + z * x

    srow = jnp.sum(h, axis=1)[None, :]
    h1 = jnp.maximum(
        lax.dot_general(srow, l1w_ref[...], (((1,), (1,)), ((), ())),
                        preferred_element_type=jnp.float32)
        + l1b_ref[...][None, :], 0.0)
    q_ref[...] = jnp.full((1, 1), jnp.sum(h1 * l2w_ref[...]) + l2b_ref[0],
                          jnp.float32)


def kernel(x, edge_index, edge_attr, gcn_W, gcn_b, W_ih, b_ih, W_hh, b_hh,
           lin1_W, lin1_b, lin2_W, lin2_b):
    del edge_attr
    a2 = _adj_kernel(edge_index)
    q = pl.pallas_call(
        _tc_body,
        out_shape=jax.ShapeDtypeStruct((1, 1), jnp.float32),
    )(a2, x, gcn_W, gcn_b, W_ih, b_ih, W_hh, b_hh, lin1_W, lin1_b, lin2_W, lin2_b)
    return q.reshape(1)


# speedup vs baseline: 31.1141x; 1.0012x over previous
import jax
import jax.numpy as jnp
from jax import lax
from jax.experimental import pallas as pl
from jax.experimental.pallas import tpu as pltpu
from jax.experimental.pallas import tpu_sc as plsc

N = 1024
D = 1024
E = 65536

NC = 2
NS = 16
HALF = N // NC
HW = HALF * N
SL = HW // NS
EPS = E // NS
GROUP = 128
NG = EPS // GROUP
ZCH = 2048
FIRE = 16


def _adj_body(ei_hbm, out_hbm, srcv, dstv, fidx, vals, zbuf, a_sh, zsem, esem, ssem):
    c = lax.axis_index("c")
    s = lax.axis_index("s")

    base = s * EPS
    eld = [
        pltpu.async_copy(ei_hbm.at[0, pl.ds(base, EPS)], srcv, esem),
        pltpu.async_copy(ei_hbm.at[1, pl.ds(base, EPS)], dstv, esem),
    ]

    def _fill_zero(i, carry):
        zbuf[pl.ds(i * 16, 16)] = jnp.zeros((16,), jnp.float32)
        return carry
    lax.fori_loop(0, ZCH // 16, _fill_zero, 0)

    zld = [
        pltpu.async_copy(zbuf, a_sh.at[pl.ds(s * SL + i * ZCH, ZCH)], zsem)
        for i in range(SL // ZCH)
    ]
    for dsc in eld:
        dsc.wait()

    def _grp(g, carry):
        for k in range(GROUP // 16):
            off = g * GROUP + k * 16
            sv = srcv[pl.ds(off, 16)]
            dv = dstv[pl.ds(off, 16)]
            mine = lax.shift_right_logical(dv, 9) == c
            loc = ((dv - c * HALF) * N + sv) & (HW - 1)
            fidx[g, pl.ds(k * 16, 16)] = loc
            vals[g, pl.ds(k * 16, 16)] = jnp.where(mine, 1.0, 0.0)
        return carry
    lax.fori_loop(0, NG, _grp, 0)

    for dsc in zld:
        dsc.wait()
    plsc.subcore_barrier()

    for lo in range(0, NG, FIRE):
        descs = [
            pltpu.async_copy(vals.at[g], a_sh.at[fidx.at[g]], ssem, add=True)
            for g in range(lo, lo + FIRE)
        ]
        for dsc in descs:
            dsc.wait()

    plsc.subcore_barrier()

    row0 = c * HALF + s * (SL // N)
    rdescs = [
        pltpu.async_copy(a_sh.at[pl.ds(s * SL + r * N, N)], out_hbm.at[row0 + r], zsem)
        for r in range(SL // N)
    ]
    for dsc in rdescs:
        dsc.wait()


_adj_kernel = pl.kernel(
    _adj_body,
    out_type=jax.ShapeDtypeStruct((N, N), jnp.float32),
    mesh=plsc.VectorSubcoreMesh(
        core_axis_name="c", subcore_axis_name="s", num_cores=NC, num_subcores=NS
    ),
    scratch_types=[
        pltpu.VMEM((EPS,), jnp.int32),
        pltpu.VMEM((EPS,), jnp.int32),
        pltpu.VMEM((NG, GROUP), jnp.int32),
        pltpu.VMEM((NG, GROUP), jnp.float32),
        pltpu.VMEM((ZCH,), jnp.float32),
        pltpu.VMEM_SHARED((HW,), jnp.float32),
        pltpu.SemaphoreType.DMA,
        pltpu.SemaphoreType.DMA,
        pltpu.SemaphoreType.DMA,
    ],
)


def _sigmoid(t):
    return 1.0 / (1.0 + jnp.exp(-t))


def _tc_body(a_ref, x_ref, gw_ref, gb_ref, wih_ref, bih_ref, whh_ref, bhh_ref,
             l1w_ref, l1b_ref, l2w_ref, l2b_ref, q_ref):
    acnt = a_ref[...]
    deg = jnp.sum(acnt, axis=1) + 1.0
    dinv = lax.rsqrt(deg)

    x = x_ref[...]
    xw = jnp.dot(x, gw_ref[...], preferred_element_type=jnp.float32)
    xs = xw * dinv[:, None]
    agg = (jnp.dot(acnt, xs, preferred_element_type=jnp.float32) + xs) * dinv[:, None]
    nf = jnp.maximum(agg + gb_ref[...][None, :], 0.0)

    bih = bih_ref[...]
    bhh = bhh_ref[...]

    def gate(inp, w_ref, lo):
        w = w_ref[pl.ds(lo, D), :]
        return lax.dot_general(inp, w, (((1,), (1,)), ((), ())),
                               preferred_element_type=jnp.float32)

    r = _sigmoid(gate(nf, wih_ref, 0) + bih[:D][None, :]
                 + gate(x, whh_ref, 0) + bhh[:D][None, :])
    z = _sigmoid(gate(nf, wih_ref, D) + bih[D:2 * D][None, :]
                 + gate(x, whh_ref, D) + bhh[D:2 * D][None, :])
    i_n = gate(nf, wih_ref, 2 * D) + bih[2 * D:][None, :]
    h_n = gate(x, whh_ref, 2 * D) + bhh[2 * D:][None, :]
    nt = jnp.tanh(i_n + r * h_n)
    h = (1.0 - z) * nt + z * x

    srow = jnp.sum(h, axis=1)[None, :]
    h1 = jnp.maximum(
        lax.dot_general(srow, l1w_ref[...], (((1,), (1,)), ((), ())),
                        preferred_element_type=jnp.float32)
        + l1b_ref[...][None, :], 0.0)
    q_ref[...] = jnp.full((1, 1), jnp.sum(h1 * l2w_ref[...]) + l2b_ref[0],
                          jnp.float32)


def kernel(x, edge_index, edge_attr, gcn_W, gcn_b, W_ih, b_ih, W_hh, b_hh,
           lin1_W, lin1_b, lin2_W, lin2_b):
    del edge_attr
    a2 = _adj_kernel(edge_index)
    q = pl.pallas_call(
        _tc_body,
        out_shape=jax.ShapeDtypeStruct((1, 1), jnp.float32),
    )(a2, x, gcn_W, gcn_b, W_ih, b_ih, W_hh, b_hh, lin1_W, lin1_b, lin2_W, lin2_b)
    return q.reshape(1)
